# baseline (speedup 1.0000x reference)
import functools

import jax
import jax.numpy as jnp
from jax import lax
from jax.experimental import pallas as pl
from jax.experimental.pallas import tpu as pltpu
from jax.experimental.pallas import tpu_sc as plsc

N = 100000
D = 128
B = 64
NW = 32
RPW = N // NW
CH = 125
NCHUNK = RPW // CH
IDS_COPY = RPW + 11
IDS_LEN = IDS_COPY + 16
NLANE = 16
NSEG = D // NLANE
BLK = 25
NBLK = CH // BLK


def _sc_partial_pool(x_flat, batch_padded):
    mesh = plsc.VectorSubcoreMesh(core_axis_name="c", subcore_axis_name="s")

    @functools.partial(
        pl.kernel,
        out_type=[
            jax.ShapeDtypeStruct((NW, B * D), jnp.float32),
            jax.ShapeDtypeStruct((NW, B * D), jnp.float32),
            jax.ShapeDtypeStruct((NW, B * NLANE), jnp.float32),
        ],
        mesh=mesh,
        compiler_params=pltpu.CompilerParams(use_tc_tiling_on_sc=False),
        scratch_types=[
            pltpu.VMEM((IDS_LEN,), jnp.int32),
            pltpu.VMEM((CH * D,), jnp.float32),
            pltpu.VMEM((CH * D,), jnp.float32),
            pltpu.VMEM((B * D,), jnp.float32),
            pltpu.VMEM((B * D,), jnp.float32),
            pltpu.VMEM((B * NLANE,), jnp.float32),
            pltpu.SemaphoreType.DMA,
            pltpu.SemaphoreType.DMA,
        ],
    )
    def sc_kernel(x_hbm, ids_hbm, psum_hbm, pmax_hbm, pcnt_hbm,
                  ids_v, xb0, xb1, sum_acc, max_acc, cnt_acc, sem0, sem1):
        cid = lax.axis_index("c")
        sid = lax.axis_index("s")
        wid = sid * 2 + cid
        start = wid * RPW
        aligned = jnp.minimum((start // 8) * 8, N - IDS_COPY)
        off = start - aligned

        bufs_sems_first = pltpu.async_copy(
            x_hbm.at[pl.ds(start * D, CH * D)], xb0, sem0)

        pltpu.sync_copy(ids_hbm.at[pl.ds(aligned, IDS_COPY)], ids_v.at[pl.ds(0, IDS_COPY)])

        def get_id(p):
            return ids_v[pl.ds(p, NLANE)][0]

        zero16 = jnp.zeros((NLANE,), jnp.float32)
        ninf16 = jnp.full((NLANE,), -jnp.inf, jnp.float32)

        def init_body(k, _):
            cnt_acc[pl.ds(k * NLANE, NLANE)] = zero16
            return 0
        lax.fori_loop(0, B, init_body, 0)

        bufs = (xb0, xb1)
        sems = (sem0, sem1)
        copies = [None, None]
        copies[0] = bufs_sems_first

        cur = get_id(off)
        cnt = jnp.float32(0.0)
        sums = [zero16] * NSEG
        maxs = [ninf16] * NSEG

        def make_row_body(xbuf, base):
            def row_body(i, carry):
                cur, cnt = carry[0], carry[1]
                sums = list(carry[2:2 + NSEG])
                maxs = list(carry[2 + NSEG:2 + 2 * NSEG])
                b = get_id(off + base + i)

                def flush(cur, cnt, *vecs):
                    for j in range(NSEG):
                        sum_acc[pl.ds(cur * D + j * NLANE, NLANE)] = vecs[j]
                        max_acc[pl.ds(cur * D + j * NLANE, NLANE)] = vecs[NSEG + j]
                    cnt_acc[pl.ds(cur * NLANE, NLANE)] = jnp.broadcast_to(cnt, (NLANE,))
                    return (jnp.float32(0.0),) + (zero16,) * NSEG + (ninf16,) * NSEG

                def keep(cur, cnt, *vecs):
                    return (cnt,) + vecs

                res = lax.cond(b != cur, flush, keep, cur, cnt, *sums, *maxs)
                cnt = res[0]
                sums = list(res[1:1 + NSEG])
                maxs = list(res[1 + NSEG:1 + 2 * NSEG])
                for j in range(NSEG):
                    r = xbuf[pl.ds(i * D + j * NLANE, NLANE)]
                    sums[j] = sums[j] + r
                    maxs[j] = jnp.maximum(maxs[j], r)
                return (b, cnt + 1.0) + tuple(sums) + tuple(maxs)
            return row_body

        def make_block_body(xbuf, base):
            row_body = make_row_body(xbuf, base)

            def block_body(blk, carry):
                cur, cnt = carry[0], carry[1]
                p = off + base + blk * BLK
                first = get_id(p)
                last = get_id(p + BLK - 1)
                uniform = jnp.logical_and(first == last, first == cur)
                ui = uniform.astype(jnp.int32)

                def body5(k, c):
                    s = list(c[:NSEG])
                    m = list(c[NSEG:])
                    r0 = blk * BLK + k * 5
                    for r in range(5):
                        for j in range(NSEG):
                            v = xbuf[pl.ds((r0 + r) * D + j * NLANE, NLANE)]
                            s[j] = s[j] + v
                            m[j] = jnp.maximum(m[j], v)
                    return tuple(s) + tuple(m)

                vecs = lax.fori_loop(0, (BLK // 5) * ui, body5, tuple(carry[2:]))
                cnt = cnt + jnp.where(uniform, float(BLK), 0.0)
                return lax.fori_loop(blk * BLK, blk * BLK + BLK * (1 - ui),
                                     row_body, (cur, cnt) + vecs)
            return block_body

        def chunk_process(xbuf, base, carry):
            cur = carry[0]
            p0 = off + base
            first = get_id(p0)
            last = get_id(p0 + CH - 1)
            uniform = jnp.logical_and(first == last, first == cur)
            ui = uniform.astype(jnp.int32)

            def body5(k, c):
                s = list(c[:NSEG])
                m = list(c[NSEG:])
                r0 = k * 5
                for r in range(5):
                    for j in range(NSEG):
                        v = xbuf[pl.ds((r0 + r) * D + j * NLANE, NLANE)]
                        s[j] = s[j] + v
                        m[j] = jnp.maximum(m[j], v)
                return tuple(s) + tuple(m)

            vecs = lax.fori_loop(0, (CH // 5) * ui, body5, tuple(carry[2:]))
            cnt = carry[1] + jnp.where(uniform, float(CH), 0.0)
            return lax.fori_loop(0, NBLK * (1 - ui),
                                 make_block_body(xbuf, base), (cur, cnt) + vecs)

        for c in range(NCHUNK):
            if c + 1 < NCHUNK:
                nb = (c + 1) % 2
                copies[nb] = pltpu.async_copy(
                    x_hbm.at[pl.ds((start + (c + 1) * CH) * D, CH * D)],
                    bufs[nb], sems[nb])
            copies[c % 2].wait()
            carry = (cur, cnt) + tuple(sums) + tuple(maxs)
            carry = chunk_process(bufs[c % 2], c * CH, carry)
            cur, cnt = carry[0], carry[1]
            sums = list(carry[2:2 + NSEG])
            maxs = list(carry[2 + NSEG:2 + 2 * NSEG])

        for j in range(NSEG):
            sum_acc[pl.ds(cur * D + j * NLANE, NLANE)] = sums[j]
            max_acc[pl.ds(cur * D + j * NLANE, NLANE)] = maxs[j]
        cnt_acc[pl.ds(cur * NLANE, NLANE)] = jnp.broadcast_to(cnt, (NLANE,))

        pltpu.sync_copy(sum_acc, psum_hbm.at[wid])
        pltpu.sync_copy(max_acc, pmax_hbm.at[wid])
        pltpu.sync_copy(cnt_acc, pcnt_hbm.at[wid])

    return sc_kernel(x_flat, batch_padded)


def _tc_combine(psum, pmax, pcnt, pw, W, b2):
    def body(ps_ref, pm_ref, pc_ref, pw_ref, w_ref, b_ref, out_ref):
        cw = pc_ref[...][:, :, :1]
        valid = cw > 0.0
        s = jnp.sum(jnp.where(valid, ps_ref[...], 0.0), axis=0)
        m = jnp.max(jnp.where(valid, pm_ref[...], -jnp.inf), axis=0)
        cnt = jnp.sum(cw, axis=0)
        mean = s / jnp.maximum(cnt, 1.0)
        pwv = pw_ref[...]
        e = jnp.exp(pwv - jnp.max(pwv))
        w = e / jnp.sum(e)
        pooled = jnp.concatenate(
            [mean * w[0, 0], m * w[0, 1], s * w[0, 2]], axis=1)
        out = lax.dot_general(pooled, w_ref[...],
                              (((1,), (1,)), ((), ())),
                              preferred_element_type=jnp.float32)
        out_ref[...] = out + b_ref[...]

    return pl.pallas_call(
        body,
        out_shape=jax.ShapeDtypeStruct((B, D), jnp.float32),
    )(psum, pmax, pcnt, pw, W, b2)


def kernel(x, batch, pool_weights, W, b):
    psum, pmax, pcnt = _sc_partial_pool(x.reshape(-1), batch.astype(jnp.int32))
    return _tc_combine(psum.reshape(NW, B, D), pmax.reshape(NW, B, D),
                       pcnt.reshape(NW, B, NLANE),
                       pool_weights.reshape(1, 3), W, b.reshape(1, D))


# speedup vs baseline: 1.0362x; 1.0362x over previous
import functools

import jax
import jax.numpy as jnp
from jax import lax
from jax.experimental import pallas as pl
from jax.experimental.pallas import tpu as pltpu
from jax.experimental.pallas import tpu_sc as plsc

N = 100000
D = 128
B = 64
NW = 32
RPW = N // NW
CH = 125
NCHUNK = RPW // CH
IDS_COPY = RPW + 11
IDS_LEN = IDS_COPY + 16
NLANE = 16
NSEG = D // NLANE
BLK = 25
NBLK = CH // BLK


def _sc_partial_pool(x_flat, batch_padded):
    mesh = plsc.VectorSubcoreMesh(core_axis_name="c", subcore_axis_name="s")

    @functools.partial(
        pl.kernel,
        out_type=[
            jax.ShapeDtypeStruct((NW, B * D), jnp.float32),
            jax.ShapeDtypeStruct((NW, B * D), jnp.float32),
            jax.ShapeDtypeStruct((NW, B * NLANE), jnp.float32),
        ],
        mesh=mesh,
        compiler_params=pltpu.CompilerParams(use_tc_tiling_on_sc=False),
        scratch_types=[
            pltpu.VMEM((IDS_LEN,), jnp.int32),
            pltpu.VMEM((CH * D,), jnp.float32),
            pltpu.VMEM((CH * D,), jnp.float32),
            pltpu.VMEM((B * D,), jnp.float32),
            pltpu.VMEM((B * D,), jnp.float32),
            pltpu.VMEM((B * NLANE,), jnp.float32),
            pltpu.SemaphoreType.DMA,
            pltpu.SemaphoreType.DMA,
            pltpu.SemaphoreType.DMA,
        ],
    )
    def sc_kernel(x_hbm, ids_hbm, psum_hbm, pmax_hbm, pcnt_hbm,
                  ids_v, xb0, xb1, sum_acc, max_acc, cnt_acc, sem0, sem1, sem2):
        cid = lax.axis_index("c")
        sid = lax.axis_index("s")
        wid = sid * 2 + cid
        start = wid * RPW
        aligned = jnp.minimum((start // 8) * 8, N - IDS_COPY)
        off = start - aligned

        bufs_sems_first = pltpu.async_copy(
            x_hbm.at[pl.ds(start * D, CH * D)], xb0, sem0)

        pltpu.sync_copy(ids_hbm.at[pl.ds(aligned, IDS_COPY)], ids_v.at[pl.ds(0, IDS_COPY)])

        def get_id(p):
            return ids_v[pl.ds(p, NLANE)][0]

        zero16 = jnp.zeros((NLANE,), jnp.float32)
        ninf16 = jnp.full((NLANE,), -jnp.inf, jnp.float32)

        def init_body(k, _):
            cnt_acc[pl.ds(k * NLANE, NLANE)] = zero16
            return 0
        lax.fori_loop(0, B, init_body, 0)

        bufs = (xb0, xb1)
        sems = (sem0, sem1)
        copies = [None, None]
        copies[0] = bufs_sems_first

        cur = get_id(off)
        cnt = jnp.float32(0.0)
        sums = [zero16] * NSEG
        maxs = [ninf16] * NSEG

        def make_row_body(xbuf, base):
            def row_body(i, carry):
                cur, cnt = carry[0], carry[1]
                sums = list(carry[2:2 + NSEG])
                maxs = list(carry[2 + NSEG:2 + 2 * NSEG])
                b = get_id(off + base + i)

                def flush(cur, cnt, *vecs):
                    for j in range(NSEG):
                        sum_acc[pl.ds(cur * D + j * NLANE, NLANE)] = vecs[j]
                        max_acc[pl.ds(cur * D + j * NLANE, NLANE)] = vecs[NSEG + j]
                    cnt_acc[pl.ds(cur * NLANE, NLANE)] = jnp.broadcast_to(cnt, (NLANE,))
                    return (jnp.float32(0.0),) + (zero16,) * NSEG + (ninf16,) * NSEG

                def keep(cur, cnt, *vecs):
                    return (cnt,) + vecs

                res = lax.cond(b != cur, flush, keep, cur, cnt, *sums, *maxs)
                cnt = res[0]
                sums = list(res[1:1 + NSEG])
                maxs = list(res[1 + NSEG:1 + 2 * NSEG])
                for j in range(NSEG):
                    r = xbuf[pl.ds(i * D + j * NLANE, NLANE)]
                    sums[j] = sums[j] + r
                    maxs[j] = jnp.maximum(maxs[j], r)
                return (b, cnt + 1.0) + tuple(sums) + tuple(maxs)
            return row_body

        def make_block_body(xbuf, base):
            row_body = make_row_body(xbuf, base)

            def block_body(blk, carry):
                cur, cnt = carry[0], carry[1]
                p = off + base + blk * BLK
                first = get_id(p)
                last = get_id(p + BLK - 1)
                uniform = jnp.logical_and(first == last, first == cur)
                ui = uniform.astype(jnp.int32)

                def body5(k, c):
                    s = list(c[:NSEG])
                    m = list(c[NSEG:])
                    r0 = blk * BLK + k * 5
                    for r in range(5):
                        for j in range(NSEG):
                            v = xbuf[pl.ds((r0 + r) * D + j * NLANE, NLANE)]
                            s[j] = s[j] + v
                            m[j] = jnp.maximum(m[j], v)
                    return tuple(s) + tuple(m)

                vecs = lax.fori_loop(0, (BLK // 5) * ui, body5, tuple(carry[2:]))
                cnt = cnt + jnp.where(uniform, float(BLK), 0.0)
                return lax.fori_loop(blk * BLK, blk * BLK + BLK * (1 - ui),
                                     row_body, (cur, cnt) + vecs)
            return block_body

        for c in range(NCHUNK):
            if c + 1 < NCHUNK:
                nb = (c + 1) % 2
                copies[nb] = pltpu.async_copy(
                    x_hbm.at[pl.ds((start + (c + 1) * CH) * D, CH * D)],
                    bufs[nb], sems[nb])
            copies[c % 2].wait()
            carry = (cur, cnt) + tuple(sums) + tuple(maxs)
            carry = lax.fori_loop(0, NBLK, make_block_body(bufs[c % 2], c * CH), carry)
            cur, cnt = carry[0], carry[1]
            sums = list(carry[2:2 + NSEG])
            maxs = list(carry[2 + NSEG:2 + 2 * NSEG])

        for j in range(NSEG):
            sum_acc[pl.ds(cur * D + j * NLANE, NLANE)] = sums[j]
            max_acc[pl.ds(cur * D + j * NLANE, NLANE)] = maxs[j]
        cnt_acc[pl.ds(cur * NLANE, NLANE)] = jnp.broadcast_to(cnt, (NLANE,))

        w0 = pltpu.async_copy(sum_acc, psum_hbm.at[wid], sem0)
        w1 = pltpu.async_copy(max_acc, pmax_hbm.at[wid], sem1)
        w2 = pltpu.async_copy(cnt_acc, pcnt_hbm.at[wid], sem2)
        w0.wait()
        w1.wait()
        w2.wait()

    return sc_kernel(x_flat, batch_padded)


def _tc_combine(psum, pmax, pcnt, pw, W, b2):
    def body(ps_ref, pm_ref, pc_ref, pw_ref, w_ref, b_ref, out_ref):
        cw = pc_ref[...][:, :, :1]
        valid = cw > 0.0
        s = jnp.sum(jnp.where(valid, ps_ref[...], 0.0), axis=0)
        m = jnp.max(jnp.where(valid, pm_ref[...], -jnp.inf), axis=0)
        cnt = jnp.sum(cw, axis=0)
        mean = s / jnp.maximum(cnt, 1.0)
        pwv = pw_ref[...]
        e = jnp.exp(pwv - jnp.max(pwv))
        w = e / jnp.sum(e)
        pooled = jnp.concatenate(
            [mean * w[0, 0], m * w[0, 1], s * w[0, 2]], axis=1)
        out = lax.dot_general(pooled, w_ref[...],
                              (((1,), (1,)), ((), ())),
                              preferred_element_type=jnp.float32)
        out_ref[...] = out + b_ref[...]

    return pl.pallas_call(
        body,
        out_shape=jax.ShapeDtypeStruct((B, D), jnp.float32),
    )(psum, pmax, pcnt, pw, W, b2)


def kernel(x, batch, pool_weights, W, b):
    psum, pmax, pcnt = _sc_partial_pool(x.reshape(-1), batch.astype(jnp.int32))
    return _tc_combine(psum.reshape(NW, B, D), pmax.reshape(NW, B, D),
                       pcnt.reshape(NW, B, NLANE),
                       pool_weights.reshape(1, 3), W, b.reshape(1, D))


# speedup vs baseline: 1.0373x; 1.0011x over previous
import functools

import jax
import jax.numpy as jnp
from jax import lax
from jax.experimental import pallas as pl
from jax.experimental.pallas import tpu as pltpu
from jax.experimental.pallas import tpu_sc as plsc

N = 100000
D = 128
B = 64
NW = 32
RPW = N // NW
CH = 125
NCHUNK = RPW // CH
IDS_COPY = RPW + 11
IDS_LEN = IDS_COPY + 16
NLANE = 16
NSEG = D // NLANE
BLK = 25
NBLK = CH // BLK


def _sc_partial_pool(x_flat, batch_padded):
    mesh = plsc.VectorSubcoreMesh(core_axis_name="c", subcore_axis_name="s")

    @functools.partial(
        pl.kernel,
        out_type=[
            jax.ShapeDtypeStruct((NW, B * D), jnp.float32),
            jax.ShapeDtypeStruct((NW, B * D), jnp.float32),
            jax.ShapeDtypeStruct((NW, B * NLANE), jnp.float32),
        ],
        mesh=mesh,
        compiler_params=pltpu.CompilerParams(use_tc_tiling_on_sc=False),
        scratch_types=[
            pltpu.VMEM((IDS_LEN,), jnp.int32),
            pltpu.VMEM((CH * D,), jnp.float32),
            pltpu.VMEM((CH * D,), jnp.float32),
            pltpu.VMEM((B * D,), jnp.float32),
            pltpu.VMEM((B * D,), jnp.float32),
            pltpu.VMEM((B * NLANE,), jnp.float32),
            pltpu.SemaphoreType.DMA,
            pltpu.SemaphoreType.DMA,
            pltpu.SemaphoreType.DMA,
        ],
    )
    def sc_kernel(x_hbm, ids_hbm, psum_hbm, pmax_hbm, pcnt_hbm,
                  ids_v, xb0, xb1, sum_acc, max_acc, cnt_acc, sem0, sem1, sem2):
        cid = lax.axis_index("c")
        sid = lax.axis_index("s")
        wid = sid * 2 + cid
        start = wid * RPW
        aligned = jnp.minimum((start // 8) * 8, N - IDS_COPY)
        off = start - aligned

        bufs_sems_first = pltpu.async_copy(
            x_hbm.at[pl.ds(start * D, CH * D)], xb0, sem0)

        pltpu.sync_copy(ids_hbm.at[pl.ds(aligned, IDS_COPY)], ids_v.at[pl.ds(0, IDS_COPY)])

        def get_id(p):
            return ids_v[pl.ds(p, NLANE)][0]

        zero16 = jnp.zeros((NLANE,), jnp.float32)
        ninf16 = jnp.full((NLANE,), -jnp.inf, jnp.float32)

        def init_body(k, _):
            cnt_acc[pl.ds(k * NLANE, NLANE)] = zero16
            return 0
        lax.fori_loop(0, B, init_body, 0)

        bufs = (xb0, xb1)
        sems = (sem0, sem1)
        copies = [None, None]
        copies[0] = bufs_sems_first

        cur = get_id(off)
        cnt = jnp.float32(0.0)
        sums = [zero16] * NSEG
        maxs = [ninf16] * NSEG

        def make_row_body(xbuf, base):
            def row_body(i, carry):
                cur, cnt = carry[0], carry[1]
                sums = list(carry[2:2 + NSEG])
                maxs = list(carry[2 + NSEG:2 + 2 * NSEG])
                b = get_id(off + base + i)

                def flush(cur, cnt, *vecs):
                    for j in range(NSEG):
                        sum_acc[pl.ds(cur * D + j * NLANE, NLANE)] = vecs[j]
                        max_acc[pl.ds(cur * D + j * NLANE, NLANE)] = vecs[NSEG + j]
                    cnt_acc[pl.ds(cur * NLANE, NLANE)] = jnp.broadcast_to(cnt, (NLANE,))
                    return (jnp.float32(0.0),) + (zero16,) * NSEG + (ninf16,) * NSEG

                def keep(cur, cnt, *vecs):
                    return (cnt,) + vecs

                res = lax.cond(b != cur, flush, keep, cur, cnt, *sums, *maxs)
                cnt = res[0]
                sums = list(res[1:1 + NSEG])
                maxs = list(res[1 + NSEG:1 + 2 * NSEG])
                for j in range(NSEG):
                    r = xbuf[pl.ds(i * D + j * NLANE, NLANE)]
                    sums[j] = sums[j] + r
                    maxs[j] = jnp.maximum(maxs[j], r)
                return (b, cnt + 1.0) + tuple(sums) + tuple(maxs)
            return row_body

        def make_block_body(xbuf, base):
            row_body = make_row_body(xbuf, base)

            def block_body(blk, carry):
                cur, cnt = carry[0], carry[1]
                p = off + base + blk * BLK
                first = get_id(p)
                last = get_id(p + BLK - 1)
                uniform = jnp.logical_and(first == last, first == cur)
                ui = uniform.astype(jnp.int32)

                def body5(k, c):
                    s = list(c[:NSEG])
                    m = list(c[NSEG:])
                    r0 = blk * BLK + k * 5
                    for r in range(5):
                        for j in range(NSEG):
                            v = xbuf[pl.ds((r0 + r) * D + j * NLANE, NLANE)]
                            s[j] = s[j] + v
                            m[j] = jnp.maximum(m[j], v)
                    return tuple(s) + tuple(m)

                vecs = lax.fori_loop(0, (BLK // 5) * ui, body5, tuple(carry[2:]))
                cnt = cnt + jnp.where(uniform, float(BLK), 0.0)
                return lax.fori_loop(blk * BLK, blk * BLK + BLK * (1 - ui),
                                     row_body, (cur, cnt) + vecs)
            return block_body

        for c in range(NCHUNK):
            if c + 1 < NCHUNK:
                nb = (c + 1) % 2
                copies[nb] = pltpu.async_copy(
                    x_hbm.at[pl.ds((start + (c + 1) * CH) * D, CH * D)],
                    bufs[nb], sems[nb])
            copies[c % 2].wait()
            carry = (cur, cnt) + tuple(sums) + tuple(maxs)
            carry = lax.fori_loop(0, NBLK, make_block_body(bufs[c % 2], c * CH), carry)
            cur, cnt = carry[0], carry[1]
            sums = list(carry[2:2 + NSEG])
            maxs = list(carry[2 + NSEG:2 + 2 * NSEG])

        for j in range(NSEG):
            sum_acc[pl.ds(cur * D + j * NLANE, NLANE)] = sums[j]
            max_acc[pl.ds(cur * D + j * NLANE, NLANE)] = maxs[j]
        cnt_acc[pl.ds(cur * NLANE, NLANE)] = jnp.broadcast_to(cnt, (NLANE,))

        w0 = pltpu.async_copy(sum_acc, psum_hbm.at[wid], sem0)
        w1 = pltpu.async_copy(max_acc, pmax_hbm.at[wid], sem1)
        w2 = pltpu.async_copy(cnt_acc, pcnt_hbm.at[wid], sem2)
        w0.wait()
        w1.wait()
        w2.wait()

    return sc_kernel(x_flat, batch_padded)


def _tc_combine(psum, pmax, pcnt, pw, W, b2):
    def body(ps_ref, pm_ref, pc_ref, pw_ref, w_ref, b_ref, out_ref):
        cw = pc_ref[...][:, :, :1]
        valid = cw > 0.0
        ps = ps_ref[...].reshape(NW, B, D)
        pm = pm_ref[...].reshape(NW, B, D)
        s = jnp.sum(jnp.where(valid, ps, 0.0), axis=0)
        m = jnp.max(jnp.where(valid, pm, -jnp.inf), axis=0)
        cnt = jnp.sum(cw, axis=0)
        mean = s / jnp.maximum(cnt, 1.0)
        pwv = pw_ref[...]
        e = jnp.exp(pwv - jnp.max(pwv))
        w = e / jnp.sum(e)
        pooled = jnp.concatenate(
            [mean * w[0, 0], m * w[0, 1], s * w[0, 2]], axis=1)
        out = lax.dot_general(pooled, w_ref[...],
                              (((1,), (1,)), ((), ())),
                              preferred_element_type=jnp.float32)
        out_ref[...] = out + b_ref[...]

    return pl.pallas_call(
        body,
        out_shape=jax.ShapeDtypeStruct((B, D), jnp.float32),
    )(psum, pmax, pcnt, pw, W, b2)


def kernel(x, batch, pool_weights, W, b):
    psum, pmax, pcnt = _sc_partial_pool(x.reshape(-1), batch.astype(jnp.int32))
    return _tc_combine(psum.reshape(NW * B, D), pmax.reshape(NW * B, D),
                       pcnt.reshape(NW, B, NLANE),
                       pool_weights.reshape(1, 3), W, b.reshape(1, D))
